# baseline (speedup 1.0000x reference)
import functools

import jax
import jax.numpy as jnp
from jax import lax
from jax.experimental import pallas as pl
from jax.experimental.pallas import tpu as pltpu
from jax.experimental.pallas import tpu_sc as plsc

N_NODES = 10000
D_FEAT = 128
D_HID = 16

N_PAD = 10240
NW = 32
IDX_W = 128
TILE_ROWS = 80
E_PAD = NW * TILE_ROWS * IDX_W
R_IDX = E_PAD // IDX_W
STRIPE = N_PAD // 16

HALF = N_PAD // 2
ACC_ROWS = 5376
ACC_STRIPE = ACC_ROWS // 16
TRASH = HALF
EPT = E_PAD // 16
CHUNKS = EPT // IDX_W

_MESH = plsc.VectorSubcoreMesh(core_axis_name="c", subcore_axis_name="s")


@functools.partial(
    pl.kernel,
    out_type=jax.ShapeDtypeStruct((2, N_PAD), jnp.float32),
    mesh=_MESH,
    scratch_types=[
        pltpu.VMEM((TILE_ROWS, IDX_W), jnp.int32),
        pltpu.VMEM((IDX_W,), jnp.float32),
        pltpu.VMEM((STRIPE,), jnp.float32),
        pltpu.VMEM_SHARED((N_PAD,), jnp.float32),
    ],
)
def _deg_kernel(dst_hbm, out_hbm, idx_v, ones_v, zv, acc_sh):
    c = lax.axis_index("c")
    s = lax.axis_index("s")
    wid = c * 16 + s

    def _zero(i, carry):
        zv[pl.ds(i * 16, 16)] = jnp.zeros((16,), jnp.float32)
        return carry

    lax.fori_loop(0, STRIPE // 16, _zero, 0)
    for i in range(IDX_W // 16):
        ones_v[pl.ds(i * 16, 16)] = jnp.ones((16,), jnp.float32)
    pltpu.sync_copy(zv, acc_sh.at[pl.ds(s * STRIPE, STRIPE)])
    pltpu.sync_copy(dst_hbm.at[pl.ds(wid * TILE_ROWS, TILE_ROWS)], idx_v)
    plsc.subcore_barrier()

    def _scat(j, carry):
        pltpu.sync_copy(ones_v, acc_sh.at[idx_v.at[j]], add=True)
        return carry

    lax.fori_loop(0, TILE_ROWS, _scat, 0)
    plsc.subcore_barrier()
    pltpu.sync_copy(acc_sh.at[pl.ds(s * STRIPE, STRIPE)],
                    out_hbm.at[c, pl.ds(s * STRIPE, STRIPE)])


@functools.partial(
    pl.kernel,
    out_type=jax.ShapeDtypeStruct((2, HALF, D_HID), jnp.float32),
    mesh=_MESH,
    compiler_params=pltpu.CompilerParams(use_tc_tiling_on_sc=False),
    scratch_types=[
        pltpu.VMEM((CHUNKS, IDX_W), jnp.int32),
        pltpu.VMEM((CHUNKS, IDX_W), jnp.int32),
        pltpu.VMEM((2, IDX_W, D_HID), jnp.float32),
        pltpu.VMEM((ACC_STRIPE // 2, D_HID), jnp.float32),
        pltpu.VMEM_SHARED((ACC_ROWS, D_HID), jnp.float32),
        pltpu.SemaphoreType.DMA,
        pltpu.SemaphoreType.DMA,
    ],
)
def _edge_scatter(tab_hbm, src_hbm, dst_hbm, out_hbm,
                  srcv, dstv, rows, zv, acc_sh, gsem, ssem):
    c = lax.axis_index("c")
    s = lax.axis_index("s")

    def _zero(i, carry):
        zv[i] = jnp.zeros((D_HID,), jnp.float32)
        return carry

    lax.fori_loop(0, ACC_STRIPE // 2, _zero, 0)
    pltpu.sync_copy(zv, acc_sh.at[pl.ds(s * ACC_STRIPE, ACC_STRIPE // 2)])
    pltpu.sync_copy(
        zv, acc_sh.at[pl.ds(s * ACC_STRIPE + ACC_STRIPE // 2,
                            ACC_STRIPE // 2)])
    pltpu.sync_copy(src_hbm.at[pl.ds(s * CHUNKS, CHUNKS)], srcv)
    pltpu.sync_copy(dst_hbm.at[c, pl.ds(s * CHUNKS, CHUNKS)], dstv)
    plsc.subcore_barrier()

    def _gather(k, b):
        pltpu.async_copy(tab_hbm.at[srcv.at[k]], rows.at[b], gsem)

    def _wait_gather(k, b):
        pltpu.make_async_copy(tab_hbm.at[srcv.at[k]], rows.at[b],
                              gsem).wait()

    def _scatter(k, b):
        pltpu.async_copy(rows.at[b], acc_sh.at[dstv.at[k]], ssem, add=True)

    def _drain_scatter(k, b):
        pltpu.make_async_copy(rows.at[b], acc_sh.at[dstv.at[k]],
                              ssem).wait()

    _gather(0, 0)
    _gather(1, 1)

    def _pair(i, carry):
        k0 = i * 2
        _wait_gather(k0, 0)
        _scatter(k0, 0)
        _wait_gather(k0 + 1, 1)
        _scatter(k0 + 1, 1)
        _drain_scatter(k0, 0)
        _drain_scatter(k0 + 1, 1)

        @pl.when(i < CHUNKS // 2 - 1)
        def _():
            _gather(k0 + 2, 0)
            _gather(k0 + 3, 1)

        return carry

    lax.fori_loop(0, CHUNKS // 2, _pair, 0)
    plsc.subcore_barrier()
    pltpu.sync_copy(acc_sh.at[pl.ds(s * (HALF // 16), HALF // 16)],
                    out_hbm.at[c, pl.ds(s * (HALF // 16), HALF // 16)])


def _tc1_body(dega_ref, degb_ref, x_ref, w1_ref, y_ref, dis_ref):
    deg = dega_ref[...] + degb_ref[...] + 1.0
    dis = lax.rsqrt(deg)
    xw = jnp.dot(x_ref[...], w1_ref[...], preferred_element_type=jnp.float32)
    y_ref[...] = dis * xw
    dis_ref[...] = dis


def _tc2_body(acc_ref, y_ref, dis_ref, b1_ref, w2_ref, t_ref):
    dis = dis_ref[...]
    pre = (acc_ref[0] + y_ref[...]) * dis + b1_ref[...]
    h = jnp.maximum(pre, 0.0)
    t_ref[...] = dis * jnp.dot(h, w2_ref[...],
                               preferred_element_type=jnp.float32)


def _tc3_body(acc_ref, t_ref, dis_ref, b2_ref, o_ref):
    o = (acc_ref[0] + t_ref[...]) * dis_ref[...] + b2_ref[...]
    m = jnp.max(o, axis=1, keepdims=True)
    e = jnp.exp(o - m)
    lse = jnp.log(jnp.sum(e, axis=1, keepdims=True)) + m
    o_ref[...] = o - lse


_GRID = 16
_BR = N_PAD // _GRID

_tc1 = pl.pallas_call(
    _tc1_body,
    grid=(_GRID,),
    in_specs=[
        pl.BlockSpec((_BR, 1), lambda i: (i, 0)),
        pl.BlockSpec((_BR, 1), lambda i: (i, 0)),
        pl.BlockSpec((_BR, D_FEAT), lambda i: (i, 0)),
        pl.BlockSpec((D_FEAT, D_HID), lambda i: (0, 0)),
    ],
    out_specs=[
        pl.BlockSpec((_BR, D_HID), lambda i: (i, 0)),
        pl.BlockSpec((_BR, 1), lambda i: (i, 0)),
    ],
    out_shape=[
        jax.ShapeDtypeStruct((N_PAD, D_HID), jnp.float32),
        jax.ShapeDtypeStruct((N_PAD, 1), jnp.float32),
    ],
)

_ACC_SPEC = pl.BlockSpec((1, _BR, D_HID), lambda i: (i // 8, i % 8, 0))

_tc2 = pl.pallas_call(
    _tc2_body,
    grid=(_GRID,),
    in_specs=[
        _ACC_SPEC,
        pl.BlockSpec((_BR, D_HID), lambda i: (i, 0)),
        pl.BlockSpec((_BR, 1), lambda i: (i, 0)),
        pl.BlockSpec((1, D_HID), lambda i: (0, 0)),
        pl.BlockSpec((D_HID, D_HID), lambda i: (0, 0)),
    ],
    out_specs=pl.BlockSpec((_BR, D_HID), lambda i: (i, 0)),
    out_shape=jax.ShapeDtypeStruct((N_PAD, D_HID), jnp.float32),
)

_tc3 = pl.pallas_call(
    _tc3_body,
    grid=(_GRID,),
    in_specs=[
        _ACC_SPEC,
        pl.BlockSpec((_BR, D_HID), lambda i: (i, 0)),
        pl.BlockSpec((_BR, 1), lambda i: (i, 0)),
        pl.BlockSpec((1, D_HID), lambda i: (0, 0)),
    ],
    out_specs=pl.BlockSpec((_BR, D_HID), lambda i: (i, 0)),
    out_shape=jax.ShapeDtypeStruct((N_PAD, D_HID), jnp.float32),
)


def kernel(x, edge_index, W1, b1, W2, b2):
    f32 = jnp.float32
    src = edge_index[0].astype(jnp.int32)
    dst = edge_index[1].astype(jnp.int32)
    pad_e = E_PAD - src.shape[0]
    pad_idx = jnp.full((pad_e,), N_NODES, jnp.int32)
    src_p = jnp.concatenate([src, pad_idx])
    dst_p = jnp.concatenate([dst, pad_idx])
    dst2 = dst_p.reshape(R_IDX, IDX_W)
    dst_a = jnp.where(dst_p < HALF, dst_p, TRASH)
    dst_b = jnp.where(dst_p >= HALF, dst_p - HALF, TRASH)
    dst_ab = jnp.stack([dst_a, dst_b]).reshape(2, R_IDX, IDX_W)
    x_p = jnp.pad(x.astype(f32), ((0, N_PAD - N_NODES), (0, 0)))

    degp = _deg_kernel(dst2)
    dega = degp[0].reshape(N_PAD, 1)
    degb = degp[1].reshape(N_PAD, 1)

    src2 = src_p.reshape(R_IDX, IDX_W)

    y1, dis = _tc1(dega, degb, x_p, W1.astype(f32))
    acc1 = _edge_scatter(y1, src2, dst_ab)
    t = _tc2(acc1, y1, dis,
             b1.reshape(1, D_HID).astype(f32), W2.astype(f32))
    acc2 = _edge_scatter(t, src2, dst_ab)
    o = _tc3(acc2, t, dis, b2.reshape(1, D_HID).astype(f32))
    return o[:N_NODES]


# speedup vs baseline: 1.0100x; 1.0100x over previous
import functools

import jax
import jax.numpy as jnp
from jax import lax
from jax.experimental import pallas as pl
from jax.experimental.pallas import tpu as pltpu
from jax.experimental.pallas import tpu_sc as plsc

N_NODES = 10000
D_FEAT = 128
D_HID = 16

N_PAD = 10240
NW = 32
IDX_W = 128
TILE_ROWS = 80
E_PAD = NW * TILE_ROWS * IDX_W
R_IDX = E_PAD // IDX_W
STRIPE = N_PAD // 16

HALF = N_PAD // 2
ACC_ROWS = 5376
ACC_STRIPE = ACC_ROWS // 16
TRASH = HALF
EPT = E_PAD // 16
CHUNKS = EPT // IDX_W

_MESH = plsc.VectorSubcoreMesh(core_axis_name="c", subcore_axis_name="s")


@functools.partial(
    pl.kernel,
    out_type=jax.ShapeDtypeStruct((2, N_PAD), jnp.float32),
    mesh=_MESH,
    scratch_types=[
        pltpu.VMEM((TILE_ROWS, IDX_W), jnp.int32),
        pltpu.VMEM((IDX_W,), jnp.float32),
        pltpu.VMEM((STRIPE,), jnp.float32),
        pltpu.VMEM_SHARED((N_PAD,), jnp.float32),
    ],
)
def _deg_kernel(dst_hbm, out_hbm, idx_v, ones_v, zv, acc_sh):
    c = lax.axis_index("c")
    s = lax.axis_index("s")
    wid = c * 16 + s

    def _zero(i, carry):
        zv[pl.ds(i * 16, 16)] = jnp.zeros((16,), jnp.float32)
        return carry

    lax.fori_loop(0, STRIPE // 16, _zero, 0)
    for i in range(IDX_W // 16):
        ones_v[pl.ds(i * 16, 16)] = jnp.ones((16,), jnp.float32)
    pltpu.sync_copy(zv, acc_sh.at[pl.ds(s * STRIPE, STRIPE)])
    pltpu.sync_copy(dst_hbm.at[pl.ds(wid * TILE_ROWS, TILE_ROWS)], idx_v)
    plsc.subcore_barrier()

    def _scat(j, carry):
        pltpu.sync_copy(ones_v, acc_sh.at[idx_v.at[j]], add=True)
        return carry

    lax.fori_loop(0, TILE_ROWS, _scat, 0)
    plsc.subcore_barrier()
    pltpu.sync_copy(acc_sh.at[pl.ds(s * STRIPE, STRIPE)],
                    out_hbm.at[c, pl.ds(s * STRIPE, STRIPE)])


@functools.partial(
    pl.kernel,
    out_type=jax.ShapeDtypeStruct((2, HALF, D_HID), jnp.float32),
    mesh=_MESH,
    compiler_params=pltpu.CompilerParams(use_tc_tiling_on_sc=False),
    scratch_types=[
        pltpu.VMEM((CHUNKS, IDX_W), jnp.int32),
        pltpu.VMEM((CHUNKS, IDX_W), jnp.int32),
        pltpu.VMEM((4, IDX_W, D_HID), jnp.float32),
        pltpu.VMEM_SHARED((ACC_ROWS, D_HID), jnp.float32),
        pltpu.SemaphoreType.DMA,
        pltpu.SemaphoreType.DMA,
    ],
)
def _edge_scatter(tab_hbm, src_hbm, dst_hbm, out_hbm,
                  srcv, dstv, rows, acc_sh, gsem, ssem):
    c = lax.axis_index("c")
    s = lax.axis_index("s")

    def _zero(i, carry):
        rows[3, i] = jnp.zeros((D_HID,), jnp.float32)
        return carry

    lax.fori_loop(0, IDX_W, _zero, 0)
    pltpu.sync_copy(rows.at[3],
                    acc_sh.at[pl.ds(s * ACC_STRIPE, IDX_W)])
    pltpu.sync_copy(rows.at[3],
                    acc_sh.at[pl.ds(s * ACC_STRIPE + IDX_W, IDX_W)])
    pltpu.sync_copy(rows.at[3, pl.ds(0, ACC_STRIPE - 2 * IDX_W)],
                    acc_sh.at[pl.ds(s * ACC_STRIPE + 2 * IDX_W,
                                    ACC_STRIPE - 2 * IDX_W)])
    pltpu.sync_copy(src_hbm.at[pl.ds(s * CHUNKS, CHUNKS)], srcv)
    pltpu.sync_copy(dst_hbm.at[c, pl.ds(s * CHUNKS, CHUNKS)], dstv)
    plsc.subcore_barrier()

    def _gather(k, b):
        pltpu.async_copy(tab_hbm.at[srcv.at[k]], rows.at[b], gsem)

    def _wait_gather(k, b):
        pltpu.make_async_copy(tab_hbm.at[srcv.at[k]], rows.at[b],
                              gsem).wait()

    def _scatter(k, b):
        pltpu.async_copy(rows.at[b], acc_sh.at[dstv.at[k]], ssem, add=True)

    def _drain_scatter(k, b):
        pltpu.make_async_copy(rows.at[b], acc_sh.at[dstv.at[k]],
                              ssem).wait()

    NB = 3
    for b in range(NB):
        _gather(b, b)

    def _span(i, carry):
        k0 = i * NB
        for b in range(NB):
            _wait_gather(k0 + b, b)
            _scatter(k0 + b, b)
        for b in range(NB):
            _drain_scatter(k0 + b, b)

            @pl.when(k0 + b + NB < CHUNKS)
            def _():
                _gather(k0 + b + NB, b)

        return carry

    lax.fori_loop(0, CHUNKS // NB, _span, 0)
    k_last = CHUNKS - 1
    _wait_gather(k_last, 0)
    _scatter(k_last, 0)
    _drain_scatter(k_last, 0)
    plsc.subcore_barrier()
    pltpu.sync_copy(acc_sh.at[pl.ds(s * (HALF // 16), HALF // 16)],
                    out_hbm.at[c, pl.ds(s * (HALF // 16), HALF // 16)])


def _tc1_body(dega_ref, degb_ref, x_ref, w1_ref, y_ref, dis_ref):
    deg = dega_ref[...] + degb_ref[...] + 1.0
    dis = lax.rsqrt(deg)
    xw = jnp.dot(x_ref[...], w1_ref[...], preferred_element_type=jnp.float32)
    y_ref[...] = dis * xw
    dis_ref[...] = dis


def _tc2_body(acc_ref, y_ref, dis_ref, b1_ref, w2_ref, t_ref):
    dis = dis_ref[...]
    pre = (acc_ref[0] + y_ref[...]) * dis + b1_ref[...]
    h = jnp.maximum(pre, 0.0)
    t_ref[...] = dis * jnp.dot(h, w2_ref[...],
                               preferred_element_type=jnp.float32)


def _tc3_body(acc_ref, t_ref, dis_ref, b2_ref, o_ref):
    o = (acc_ref[0] + t_ref[...]) * dis_ref[...] + b2_ref[...]
    m = jnp.max(o, axis=1, keepdims=True)
    e = jnp.exp(o - m)
    lse = jnp.log(jnp.sum(e, axis=1, keepdims=True)) + m
    o_ref[...] = o - lse


_GRID = 16
_BR = N_PAD // _GRID

_tc1 = pl.pallas_call(
    _tc1_body,
    grid=(_GRID,),
    in_specs=[
        pl.BlockSpec((_BR, 1), lambda i: (i, 0)),
        pl.BlockSpec((_BR, 1), lambda i: (i, 0)),
        pl.BlockSpec((_BR, D_FEAT), lambda i: (i, 0)),
        pl.BlockSpec((D_FEAT, D_HID), lambda i: (0, 0)),
    ],
    out_specs=[
        pl.BlockSpec((_BR, D_HID), lambda i: (i, 0)),
        pl.BlockSpec((_BR, 1), lambda i: (i, 0)),
    ],
    out_shape=[
        jax.ShapeDtypeStruct((N_PAD, D_HID), jnp.float32),
        jax.ShapeDtypeStruct((N_PAD, 1), jnp.float32),
    ],
)

_ACC_SPEC = pl.BlockSpec((1, _BR, D_HID), lambda i: (i // 8, i % 8, 0))

_tc2 = pl.pallas_call(
    _tc2_body,
    grid=(_GRID,),
    in_specs=[
        _ACC_SPEC,
        pl.BlockSpec((_BR, D_HID), lambda i: (i, 0)),
        pl.BlockSpec((_BR, 1), lambda i: (i, 0)),
        pl.BlockSpec((1, D_HID), lambda i: (0, 0)),
        pl.BlockSpec((D_HID, D_HID), lambda i: (0, 0)),
    ],
    out_specs=pl.BlockSpec((_BR, D_HID), lambda i: (i, 0)),
    out_shape=jax.ShapeDtypeStruct((N_PAD, D_HID), jnp.float32),
)

_tc3 = pl.pallas_call(
    _tc3_body,
    grid=(_GRID,),
    in_specs=[
        _ACC_SPEC,
        pl.BlockSpec((_BR, D_HID), lambda i: (i, 0)),
        pl.BlockSpec((_BR, 1), lambda i: (i, 0)),
        pl.BlockSpec((1, D_HID), lambda i: (0, 0)),
    ],
    out_specs=pl.BlockSpec((_BR, D_HID), lambda i: (i, 0)),
    out_shape=jax.ShapeDtypeStruct((N_PAD, D_HID), jnp.float32),
)


def kernel(x, edge_index, W1, b1, W2, b2):
    f32 = jnp.float32
    src = edge_index[0].astype(jnp.int32)
    dst = edge_index[1].astype(jnp.int32)
    pad_e = E_PAD - src.shape[0]
    pad_idx = jnp.full((pad_e,), N_NODES, jnp.int32)
    src_p = jnp.concatenate([src, pad_idx])
    dst_p = jnp.concatenate([dst, pad_idx])
    dst2 = dst_p.reshape(R_IDX, IDX_W)
    dst_a = jnp.where(dst_p < HALF, dst_p, TRASH)
    dst_b = jnp.where(dst_p >= HALF, dst_p - HALF, TRASH)
    dst_ab = jnp.stack([dst_a, dst_b]).reshape(2, R_IDX, IDX_W)
    x_p = jnp.pad(x.astype(f32), ((0, N_PAD - N_NODES), (0, 0)))

    degp = _deg_kernel(dst2)
    dega = degp[0].reshape(N_PAD, 1)
    degb = degp[1].reshape(N_PAD, 1)

    src2 = src_p.reshape(R_IDX, IDX_W)

    y1, dis = _tc1(dega, degb, x_p, W1.astype(f32))
    acc1 = _edge_scatter(y1, src2, dst_ab)
    t = _tc2(acc1, y1, dis,
             b1.reshape(1, D_HID).astype(f32), W2.astype(f32))
    acc2 = _edge_scatter(t, src2, dst_ab)
    o = _tc3(acc2, t, dis, b2.reshape(1, D_HID).astype(f32))
    return o[:N_NODES]


# speedup vs baseline: 1.0931x; 1.0822x over previous
import functools

import jax
import jax.numpy as jnp
from jax import lax
from jax.experimental import pallas as pl
from jax.experimental.pallas import tpu as pltpu
from jax.experimental.pallas import tpu_sc as plsc

N_NODES = 10000
D_FEAT = 128
D_HID = 16

N_PAD = 10240
NW = 32
IDX_W = 128
TILE_ROWS = 80
E_PAD = NW * TILE_ROWS * IDX_W
R_IDX = E_PAD // IDX_W
STRIPE = N_PAD // 16

HALF = N_PAD // 2
ACC_ROWS = 5376
ACC_STRIPE = ACC_ROWS // 16
TRASH = HALF
EPT = E_PAD // 16
CHUNKS = EPT // IDX_W

_MESH = plsc.VectorSubcoreMesh(core_axis_name="c", subcore_axis_name="s")


@functools.partial(
    pl.kernel,
    out_type=jax.ShapeDtypeStruct((2, N_PAD), jnp.float32),
    mesh=_MESH,
    scratch_types=[
        pltpu.VMEM((TILE_ROWS, IDX_W), jnp.int32),
        pltpu.VMEM((IDX_W,), jnp.float32),
        pltpu.VMEM((STRIPE,), jnp.float32),
        pltpu.VMEM_SHARED((N_PAD,), jnp.float32),
    ],
)
def _deg_kernel(dst_hbm, out_hbm, idx_v, ones_v, zv, acc_sh):
    c = lax.axis_index("c")
    s = lax.axis_index("s")
    wid = c * 16 + s

    def _zero(i, carry):
        zv[pl.ds(i * 16, 16)] = jnp.zeros((16,), jnp.float32)
        return carry

    lax.fori_loop(0, STRIPE // 16, _zero, 0)
    for i in range(IDX_W // 16):
        ones_v[pl.ds(i * 16, 16)] = jnp.ones((16,), jnp.float32)
    pltpu.sync_copy(zv, acc_sh.at[pl.ds(s * STRIPE, STRIPE)])
    pltpu.sync_copy(dst_hbm.at[pl.ds(wid * TILE_ROWS, TILE_ROWS)], idx_v)
    plsc.subcore_barrier()

    def _scat(j, carry):
        pltpu.sync_copy(ones_v, acc_sh.at[idx_v.at[j]], add=True)
        return carry

    lax.fori_loop(0, TILE_ROWS, _scat, 0)
    plsc.subcore_barrier()
    pltpu.sync_copy(acc_sh.at[pl.ds(s * STRIPE, STRIPE)],
                    out_hbm.at[c, pl.ds(s * STRIPE, STRIPE)])


@functools.partial(
    pl.kernel,
    out_type=jax.ShapeDtypeStruct((2, HALF, D_HID), jnp.float32),
    mesh=_MESH,
    compiler_params=pltpu.CompilerParams(use_tc_tiling_on_sc=False),
    scratch_types=[
        pltpu.VMEM((CHUNKS, IDX_W), jnp.int32),
        pltpu.VMEM((CHUNKS, IDX_W), jnp.int32),
        pltpu.VMEM((4, IDX_W, D_HID), jnp.float32),
        pltpu.VMEM_SHARED((N_PAD, D_HID), jnp.float32),
        pltpu.VMEM_SHARED((ACC_ROWS, D_HID), jnp.float32),
        pltpu.SemaphoreType.DMA,
        pltpu.SemaphoreType.DMA,
    ],
)
def _edge_scatter(tab_hbm, src_hbm, dst_hbm, out_hbm,
                  srcv, dstv, rows, tab_sh, acc_sh, gsem, ssem):
    c = lax.axis_index("c")
    s = lax.axis_index("s")

    def _zero(i, carry):
        rows[3, i] = jnp.zeros((D_HID,), jnp.float32)
        return carry

    lax.fori_loop(0, IDX_W, _zero, 0)
    pltpu.sync_copy(rows.at[3],
                    acc_sh.at[pl.ds(s * ACC_STRIPE, IDX_W)])
    pltpu.sync_copy(rows.at[3],
                    acc_sh.at[pl.ds(s * ACC_STRIPE + IDX_W, IDX_W)])
    pltpu.sync_copy(rows.at[3, pl.ds(0, ACC_STRIPE - 2 * IDX_W)],
                    acc_sh.at[pl.ds(s * ACC_STRIPE + 2 * IDX_W,
                                    ACC_STRIPE - 2 * IDX_W)])
    pltpu.sync_copy(src_hbm.at[pl.ds(s * CHUNKS, CHUNKS)], srcv)
    pltpu.sync_copy(dst_hbm.at[c, pl.ds(s * CHUNKS, CHUNKS)], dstv)
    pltpu.sync_copy(tab_hbm.at[pl.ds(s * (N_PAD // 16), N_PAD // 16)],
                    tab_sh.at[pl.ds(s * (N_PAD // 16), N_PAD // 16)])
    plsc.subcore_barrier()

    def _gather(k, b):
        pltpu.async_copy(tab_sh.at[srcv.at[k]], rows.at[b], gsem)

    def _wait_gather(k, b):
        pltpu.make_async_copy(tab_sh.at[srcv.at[k]], rows.at[b],
                              gsem).wait()

    def _scatter(k, b):
        pltpu.async_copy(rows.at[b], acc_sh.at[dstv.at[k]], ssem, add=True)

    def _drain_scatter(k, b):
        pltpu.make_async_copy(rows.at[b], acc_sh.at[dstv.at[k]],
                              ssem).wait()

    NB = 3
    for b in range(NB):
        _gather(b, b)

    def _span(i, carry):
        k0 = i * NB
        for b in range(NB):
            _wait_gather(k0 + b, b)
            _scatter(k0 + b, b)
        for b in range(NB):
            _drain_scatter(k0 + b, b)

            @pl.when(k0 + b + NB < CHUNKS)
            def _():
                _gather(k0 + b + NB, b)

        return carry

    lax.fori_loop(0, CHUNKS // NB, _span, 0)
    k_last = CHUNKS - 1
    _wait_gather(k_last, 0)
    _scatter(k_last, 0)
    _drain_scatter(k_last, 0)
    plsc.subcore_barrier()
    pltpu.sync_copy(acc_sh.at[pl.ds(s * (HALF // 16), HALF // 16)],
                    out_hbm.at[c, pl.ds(s * (HALF // 16), HALF // 16)])


def _tc1_body(dega_ref, degb_ref, x_ref, w1_ref, y_ref, dis_ref):
    deg = dega_ref[...] + degb_ref[...] + 1.0
    dis = lax.rsqrt(deg)
    xw = jnp.dot(x_ref[...], w1_ref[...], preferred_element_type=jnp.float32)
    y_ref[...] = dis * xw
    dis_ref[...] = dis


def _tc2_body(acc_ref, y_ref, dis_ref, b1_ref, w2_ref, t_ref):
    dis = dis_ref[...]
    pre = (acc_ref[0] + y_ref[...]) * dis + b1_ref[...]
    h = jnp.maximum(pre, 0.0)
    t_ref[...] = dis * jnp.dot(h, w2_ref[...],
                               preferred_element_type=jnp.float32)


def _tc3_body(acc_ref, t_ref, dis_ref, b2_ref, o_ref):
    o = (acc_ref[0] + t_ref[...]) * dis_ref[...] + b2_ref[...]
    m = jnp.max(o, axis=1, keepdims=True)
    e = jnp.exp(o - m)
    lse = jnp.log(jnp.sum(e, axis=1, keepdims=True)) + m
    o_ref[...] = o - lse


_GRID = 16
_BR = N_PAD // _GRID

_tc1 = pl.pallas_call(
    _tc1_body,
    grid=(_GRID,),
    in_specs=[
        pl.BlockSpec((_BR, 1), lambda i: (i, 0)),
        pl.BlockSpec((_BR, 1), lambda i: (i, 0)),
        pl.BlockSpec((_BR, D_FEAT), lambda i: (i, 0)),
        pl.BlockSpec((D_FEAT, D_HID), lambda i: (0, 0)),
    ],
    out_specs=[
        pl.BlockSpec((_BR, D_HID), lambda i: (i, 0)),
        pl.BlockSpec((_BR, 1), lambda i: (i, 0)),
    ],
    out_shape=[
        jax.ShapeDtypeStruct((N_PAD, D_HID), jnp.float32),
        jax.ShapeDtypeStruct((N_PAD, 1), jnp.float32),
    ],
)

_ACC_SPEC = pl.BlockSpec((1, _BR, D_HID), lambda i: (i // 8, i % 8, 0))

_tc2 = pl.pallas_call(
    _tc2_body,
    grid=(_GRID,),
    in_specs=[
        _ACC_SPEC,
        pl.BlockSpec((_BR, D_HID), lambda i: (i, 0)),
        pl.BlockSpec((_BR, 1), lambda i: (i, 0)),
        pl.BlockSpec((1, D_HID), lambda i: (0, 0)),
        pl.BlockSpec((D_HID, D_HID), lambda i: (0, 0)),
    ],
    out_specs=pl.BlockSpec((_BR, D_HID), lambda i: (i, 0)),
    out_shape=jax.ShapeDtypeStruct((N_PAD, D_HID), jnp.float32),
)

_tc3 = pl.pallas_call(
    _tc3_body,
    grid=(_GRID,),
    in_specs=[
        _ACC_SPEC,
        pl.BlockSpec((_BR, D_HID), lambda i: (i, 0)),
        pl.BlockSpec((_BR, 1), lambda i: (i, 0)),
        pl.BlockSpec((1, D_HID), lambda i: (0, 0)),
    ],
    out_specs=pl.BlockSpec((_BR, D_HID), lambda i: (i, 0)),
    out_shape=jax.ShapeDtypeStruct((N_PAD, D_HID), jnp.float32),
)


def kernel(x, edge_index, W1, b1, W2, b2):
    f32 = jnp.float32
    src = edge_index[0].astype(jnp.int32)
    dst = edge_index[1].astype(jnp.int32)
    pad_e = E_PAD - src.shape[0]
    pad_idx = jnp.full((pad_e,), N_NODES, jnp.int32)
    src_p = jnp.concatenate([src, pad_idx])
    dst_p = jnp.concatenate([dst, pad_idx])
    dst2 = dst_p.reshape(R_IDX, IDX_W)
    dst_a = jnp.where(dst_p < HALF, dst_p, TRASH)
    dst_b = jnp.where(dst_p >= HALF, dst_p - HALF, TRASH)
    dst_ab = jnp.stack([dst_a, dst_b]).reshape(2, R_IDX, IDX_W)
    x_p = jnp.pad(x.astype(f32), ((0, N_PAD - N_NODES), (0, 0)))

    degp = _deg_kernel(dst2)
    dega = degp[0].reshape(N_PAD, 1)
    degb = degp[1].reshape(N_PAD, 1)

    src2 = src_p.reshape(R_IDX, IDX_W)

    y1, dis = _tc1(dega, degb, x_p, W1.astype(f32))
    acc1 = _edge_scatter(y1, src2, dst_ab)
    t = _tc2(acc1, y1, dis,
             b1.reshape(1, D_HID).astype(f32), W2.astype(f32))
    acc2 = _edge_scatter(t, src2, dst_ab)
    o = _tc3(acc2, t, dis, b2.reshape(1, D_HID).astype(f32))
    return o[:N_NODES]


# speedup vs baseline: 2.8472x; 2.6048x over previous
import functools

import jax
import jax.numpy as jnp
from jax import lax
from jax.experimental import pallas as pl
from jax.experimental.pallas import tpu as pltpu
from jax.experimental.pallas import tpu_sc as plsc

N_NODES = 10000
D_FEAT = 128
D_HID = 16

N_PAD = 10240
NW = 32
IDX_W = 128
TILE_ROWS = 80
E_PAD = NW * TILE_ROWS * IDX_W
R_IDX = E_PAD // IDX_W
STRIPE = N_PAD // 16

TCHUNKS = R_IDX // NW
NBANK = 4

_MESH = plsc.VectorSubcoreMesh(core_axis_name="c", subcore_axis_name="s")


@functools.partial(
    pl.kernel,
    out_type=jax.ShapeDtypeStruct((2, N_PAD), jnp.float32),
    mesh=_MESH,
    scratch_types=[
        pltpu.VMEM((TILE_ROWS, IDX_W), jnp.int32),
        pltpu.VMEM((IDX_W,), jnp.float32),
        pltpu.VMEM((STRIPE,), jnp.float32),
        pltpu.VMEM_SHARED((N_PAD,), jnp.float32),
    ],
)
def _deg_kernel(dst_hbm, out_hbm, idx_v, ones_v, zv, acc_sh):
    c = lax.axis_index("c")
    s = lax.axis_index("s")
    wid = c * 16 + s

    def _zero(i, carry):
        zv[pl.ds(i * 16, 16)] = jnp.zeros((16,), jnp.float32)
        return carry

    lax.fori_loop(0, STRIPE // 16, _zero, 0)
    for i in range(IDX_W // 16):
        ones_v[pl.ds(i * 16, 16)] = jnp.ones((16,), jnp.float32)
    pltpu.sync_copy(zv, acc_sh.at[pl.ds(s * STRIPE, STRIPE)])
    pltpu.sync_copy(dst_hbm.at[pl.ds(wid * TILE_ROWS, TILE_ROWS)], idx_v)
    plsc.subcore_barrier()

    def _scat(j, carry):
        pltpu.sync_copy(ones_v, acc_sh.at[idx_v.at[j]], add=True)
        return carry

    lax.fori_loop(0, TILE_ROWS, _scat, 0)
    plsc.subcore_barrier()
    pltpu.sync_copy(acc_sh.at[pl.ds(s * STRIPE, STRIPE)],
                    out_hbm.at[c, pl.ds(s * STRIPE, STRIPE)])


@functools.partial(
    pl.kernel,
    out_type=jax.ShapeDtypeStruct((2, N_PAD, D_HID), jnp.float32),
    mesh=_MESH,
    compiler_params=pltpu.CompilerParams(use_tc_tiling_on_sc=False),
    scratch_types=[
        pltpu.VMEM((TCHUNKS, IDX_W), jnp.int32),
        pltpu.VMEM((TCHUNKS, IDX_W), jnp.int32),
        pltpu.VMEM((NBANK, IDX_W, D_HID), jnp.float32),
        pltpu.VMEM_SHARED((N_PAD, D_HID), jnp.float32),
        pltpu.VMEM_SHARED((N_PAD, D_HID), jnp.float32),
        pltpu.SemaphoreType.DMA,
        pltpu.SemaphoreType.DMA,
    ],
)
def _edge_scatter(tab_hbm, src_hbm, dst_hbm, out_hbm,
                  srcv, dstv, rows, tab_sh, acc_sh, gsem, ssem):
    c = lax.axis_index("c")
    s = lax.axis_index("s")
    wid = c * 16 + s
    stripe = N_PAD // 16

    def _zero(i, carry):
        rows[NBANK - 1, i] = jnp.zeros((D_HID,), jnp.float32)
        return carry

    lax.fori_loop(0, IDX_W, _zero, 0)

    def _zcopy(i, carry):
        pltpu.sync_copy(rows.at[NBANK - 1],
                        acc_sh.at[pl.ds(s * stripe + i * IDX_W, IDX_W)])
        return carry

    lax.fori_loop(0, stripe // IDX_W, _zcopy, 0)
    pltpu.sync_copy(src_hbm.at[pl.ds(wid * TCHUNKS, TCHUNKS)], srcv)
    pltpu.sync_copy(dst_hbm.at[pl.ds(wid * TCHUNKS, TCHUNKS)], dstv)
    pltpu.sync_copy(tab_hbm.at[pl.ds(s * stripe, stripe)],
                    tab_sh.at[pl.ds(s * stripe, stripe)])
    plsc.subcore_barrier()

    def _gather(k, b):
        pltpu.async_copy(tab_sh.at[srcv.at[k]], rows.at[b], gsem)

    def _wait_gather(k, b):
        pltpu.make_async_copy(tab_sh.at[srcv.at[k]], rows.at[b],
                              gsem).wait()

    def _scatter(k, b):
        pltpu.async_copy(rows.at[b], acc_sh.at[dstv.at[k]], ssem, add=True)

    def _drain_scatter(k, b):
        pltpu.make_async_copy(rows.at[b], acc_sh.at[dstv.at[k]],
                              ssem).wait()

    for b in range(NBANK):
        _gather(b, b)

    def _span(i, carry):
        k0 = i * NBANK
        for b in range(NBANK):
            _wait_gather(k0 + b, b)
            _scatter(k0 + b, b)
        for b in range(NBANK):
            _drain_scatter(k0 + b, b)

            @pl.when(k0 + b + NBANK < TCHUNKS)
            def _():
                _gather(k0 + b + NBANK, b)

        return carry

    lax.fori_loop(0, TCHUNKS // NBANK, _span, 0)
    plsc.subcore_barrier()
    pltpu.sync_copy(acc_sh.at[pl.ds(s * stripe, stripe)],
                    out_hbm.at[c, pl.ds(s * stripe, stripe)])


def _tc1_body(dega_ref, degb_ref, x_ref, w1_ref, y_ref, dis_ref):
    deg = dega_ref[...] + degb_ref[...] + 1.0
    dis = lax.rsqrt(deg)
    xw = jnp.dot(x_ref[...], w1_ref[...], preferred_element_type=jnp.float32)
    y_ref[...] = dis * xw
    dis_ref[...] = dis


def _tc2_body(aa_ref, ab_ref, y_ref, dis_ref, b1_ref, w2_ref, t_ref):
    dis = dis_ref[...]
    pre = (aa_ref[...] + ab_ref[...] + y_ref[...]) * dis + b1_ref[...]
    h = jnp.maximum(pre, 0.0)
    t_ref[...] = dis * jnp.dot(h, w2_ref[...],
                               preferred_element_type=jnp.float32)


def _tc3_body(aa_ref, ab_ref, t_ref, dis_ref, b2_ref, o_ref):
    o = (aa_ref[...] + ab_ref[...] + t_ref[...]) * dis_ref[...] + b2_ref[...]
    m = jnp.max(o, axis=1, keepdims=True)
    e = jnp.exp(o - m)
    lse = jnp.log(jnp.sum(e, axis=1, keepdims=True)) + m
    o_ref[...] = o - lse


_GRID = 16
_BR = N_PAD // _GRID

_tc1 = pl.pallas_call(
    _tc1_body,
    grid=(_GRID,),
    in_specs=[
        pl.BlockSpec((_BR, 1), lambda i: (i, 0)),
        pl.BlockSpec((_BR, 1), lambda i: (i, 0)),
        pl.BlockSpec((_BR, D_FEAT), lambda i: (i, 0)),
        pl.BlockSpec((D_FEAT, D_HID), lambda i: (0, 0)),
    ],
    out_specs=[
        pl.BlockSpec((_BR, D_HID), lambda i: (i, 0)),
        pl.BlockSpec((_BR, 1), lambda i: (i, 0)),
    ],
    out_shape=[
        jax.ShapeDtypeStruct((N_PAD, D_HID), jnp.float32),
        jax.ShapeDtypeStruct((N_PAD, 1), jnp.float32),
    ],
)

_tc2 = pl.pallas_call(
    _tc2_body,
    grid=(_GRID,),
    in_specs=[
        pl.BlockSpec((_BR, D_HID), lambda i: (i, 0)),
        pl.BlockSpec((_BR, D_HID), lambda i: (i, 0)),
        pl.BlockSpec((_BR, D_HID), lambda i: (i, 0)),
        pl.BlockSpec((_BR, 1), lambda i: (i, 0)),
        pl.BlockSpec((1, D_HID), lambda i: (0, 0)),
        pl.BlockSpec((D_HID, D_HID), lambda i: (0, 0)),
    ],
    out_specs=pl.BlockSpec((_BR, D_HID), lambda i: (i, 0)),
    out_shape=jax.ShapeDtypeStruct((N_PAD, D_HID), jnp.float32),
)

_tc3 = pl.pallas_call(
    _tc3_body,
    grid=(_GRID,),
    in_specs=[
        pl.BlockSpec((_BR, D_HID), lambda i: (i, 0)),
        pl.BlockSpec((_BR, D_HID), lambda i: (i, 0)),
        pl.BlockSpec((_BR, D_HID), lambda i: (i, 0)),
        pl.BlockSpec((_BR, 1), lambda i: (i, 0)),
        pl.BlockSpec((1, D_HID), lambda i: (0, 0)),
    ],
    out_specs=pl.BlockSpec((_BR, D_HID), lambda i: (i, 0)),
    out_shape=jax.ShapeDtypeStruct((N_PAD, D_HID), jnp.float32),
)


def kernel(x, edge_index, W1, b1, W2, b2):
    f32 = jnp.float32
    src = edge_index[0].astype(jnp.int32)
    dst = edge_index[1].astype(jnp.int32)
    pad_e = E_PAD - src.shape[0]
    pad_idx = jnp.full((pad_e,), N_NODES, jnp.int32)
    src_p = jnp.concatenate([src, pad_idx])
    dst_p = jnp.concatenate([dst, pad_idx])
    dst2 = dst_p.reshape(R_IDX, IDX_W)
    x_p = jnp.pad(x.astype(f32), ((0, N_PAD - N_NODES), (0, 0)))

    degp = _deg_kernel(dst2)
    dega = degp[0].reshape(N_PAD, 1)
    degb = degp[1].reshape(N_PAD, 1)

    src2 = src_p.reshape(R_IDX, IDX_W)

    y1, dis = _tc1(dega, degb, x_p, W1.astype(f32))
    acc1 = _edge_scatter(y1, src2, dst2)
    t = _tc2(acc1[0], acc1[1], y1, dis,
             b1.reshape(1, D_HID).astype(f32), W2.astype(f32))
    acc2 = _edge_scatter(t, src2, dst2)
    o = _tc3(acc2[0], acc2[1], t, dis, b2.reshape(1, D_HID).astype(f32))
    return o[:N_NODES]
